# baseline (speedup 1.0000x reference)
import functools

import jax
import jax.numpy as jnp
from jax import lax
from jax.experimental import pallas as pl
from jax.experimental.pallas import tpu as pltpu
from jax.experimental.pallas import tpu_sc as plsc

N = 10000
D = 128
E = 320000
CH = 128
ROW_BLK = 1000


@functools.lru_cache(maxsize=None)
def _build():
    mesh = plsc.VectorSubcoreMesh(core_axis_name="c", subcore_axis_name="s")
    NC, NS = mesh.num_cores, mesh.num_subcores
    NW = NC * NS
    NCHUNK = -(-E // (NW * CH))
    NCHUNK = ((NCHUNK + 3) // 4) * 4
    EPAD = NW * NCHUNK * CH
    ACC_ROWS = ((N + 16 * NS - 1) // (16 * NS)) * (16 * NS)
    RPT = ACC_ROWS // NS
    NFULL = N // RPT
    NREM = N - NFULL * RPT
    NPAD1 = ACC_ROWS
    ZB = NPAD1 // NS

    @functools.partial(
        pl.kernel,
        out_type=jax.ShapeDtypeStruct((NC * NPAD1,), jnp.float32),
        mesh=mesh,
        scratch_types=[
            pltpu.VMEM((NCHUNK, CH), jnp.int32),
            pltpu.VMEM((CH,), jnp.float32),
            pltpu.VMEM((ZB,), jnp.float32),
            pltpu.VMEM_SHARED((NPAD1,), jnp.float32),
        ],
    )
    def sc_deg(dst_hbm, out_hbm, dst_v, ones_v, zb_v, dacc):
        cid = lax.axis_index("c")
        sid = lax.axis_index("s")
        tid = cid * NS + sid
        pltpu.sync_copy(dst_hbm.at[tid], dst_v)

        @pl.loop(0, CH, step=16)
        def _(i):
            ones_v[pl.ds(pl.multiple_of(i, 16), 16)] = jnp.full(
                (16,), 1.0, jnp.float32)

        @pl.loop(0, ZB, step=16)
        def _(i):
            zb_v[pl.ds(pl.multiple_of(i, 16), 16)] = jnp.zeros(
                (16,), jnp.float32)

        pltpu.sync_copy(zb_v, dacc.at[pl.ds(sid * ZB, ZB)])
        plsc.subcore_barrier()

        @pl.loop(0, NCHUNK)
        def _(k):
            pltpu.sync_copy(ones_v, dacc.at[dst_v.at[k]], add=True)

        plsc.subcore_barrier()
        pltpu.sync_copy(dacc.at[pl.ds(sid * ZB, ZB)],
                        out_hbm.at[pl.ds(cid * NPAD1 + sid * ZB, ZB)])

    NPH = 2
    assert NCHUNK % (2 * NPH) == 0 and RPT % CH == 0
    HC = NCHUNK // NPH

    @functools.partial(
        pl.kernel,
        out_type=jax.ShapeDtypeStruct((NC, ACC_ROWS, D), jnp.float32),
        mesh=mesh,
        scratch_types=[
            pltpu.VMEM((HC, CH), jnp.int32),
            pltpu.VMEM((HC, CH), jnp.int32),
            pltpu.VMEM((CH, D), jnp.float32),
            pltpu.VMEM((CH, D), jnp.float32),
            pltpu.VMEM_SHARED((ACC_ROWS, D), jnp.float32),
            pltpu.SemaphoreType.DMA,
            pltpu.SemaphoreType.DMA,
        ],
    )
    def sc_agg(y_hbm, src_hbm, dst_hbm, out_hbm,
               src_v, dst_v, rows_a, rows_b, acc, sga, sgb):
        cid = lax.axis_index("c")
        sid = lax.axis_index("s")
        tid = cid * NS + sid
        r0 = sid * RPT

        @pl.loop(0, CH)
        def _(r):
            for c in range(0, D, 16):
                rows_a[r, pl.ds(c, 16)] = jnp.zeros((16,), jnp.float32)

        @pl.loop(0, RPT, step=CH)
        def _(r):
            pltpu.sync_copy(rows_a, acc.at[pl.ds(r0 + r, CH)])

        plsc.subcore_barrier()

        for ph in range(NPH):
            pltpu.sync_copy(src_hbm.at[tid, pl.ds(ph * HC, HC)], src_v)
            pltpu.sync_copy(dst_hbm.at[tid, pl.ds(ph * HC, HC)], dst_v)
            @pl.loop(0, HC)
            def _(k):
                pltpu.sync_copy(y_hbm.at[src_v.at[k]], rows_a)
                pltpu.sync_copy(rows_a, acc.at[dst_v.at[k]], add=True)

        plsc.subcore_barrier()
        pltpu.sync_copy(acc.at[pl.ds(r0, RPT)],
                        out_hbm.at[cid, pl.ds(r0, RPT)])

    dot = functools.partial(
        lax.dot_general,
        dimension_numbers=(((1,), (0,)), ((), ())),
        precision=lax.Precision.HIGHEST,
        preferred_element_type=jnp.float32,
    )

    def m1_body(x_ref, w_ref, dinv_ref, o_ref):
        o_ref[...] = dot(x_ref[...], w_ref[...]) * dinv_ref[...]

    tc_m1 = pl.pallas_call(
        m1_body,
        grid=(N // ROW_BLK,),
        in_specs=[
            pl.BlockSpec((ROW_BLK, D), lambda i: (i, 0)),
            pl.BlockSpec((D, D), lambda i: (0, 0)),
            pl.BlockSpec((ROW_BLK, 1), lambda i: (i, 0)),
        ],
        out_specs=pl.BlockSpec((ROW_BLK, D), lambda i: (i, 0)),
        out_shape=jax.ShapeDtypeStruct((N, D), jnp.float32),
    )

    def m2_body(s_ref, y_ref, dinv_ref, b_ref, w_ref, o_ref):
        h = (s_ref[0] + s_ref[1] + y_ref[...]) * dinv_ref[...] + b_ref[...]
        h = jnp.maximum(h, 0.0)
        o_ref[...] = dot(h, w_ref[...]) * dinv_ref[...]

    tc_m2 = pl.pallas_call(
        m2_body,
        grid=(N // ROW_BLK,),
        in_specs=[
            pl.BlockSpec((NC, ROW_BLK, D), lambda i: (0, i, 0)),
            pl.BlockSpec((ROW_BLK, D), lambda i: (i, 0)),
            pl.BlockSpec((ROW_BLK, 1), lambda i: (i, 0)),
            pl.BlockSpec((1, D), lambda i: (0, 0)),
            pl.BlockSpec((D, D), lambda i: (0, 0)),
        ],
        out_specs=pl.BlockSpec((ROW_BLK, D), lambda i: (i, 0)),
        out_shape=jax.ShapeDtypeStruct((N, D), jnp.float32),
    )

    def ep_body(s_ref, y_ref, dinv_ref, b_ref, o_ref):
        o_ref[...] = (s_ref[0] + s_ref[1] + y_ref[...]) * dinv_ref[...] \
            + b_ref[...]

    tc_ep = pl.pallas_call(
        ep_body,
        grid=(N // ROW_BLK,),
        in_specs=[
            pl.BlockSpec((NC, ROW_BLK, D), lambda i: (0, i, 0)),
            pl.BlockSpec((ROW_BLK, D), lambda i: (i, 0)),
            pl.BlockSpec((ROW_BLK, 1), lambda i: (i, 0)),
            pl.BlockSpec((1, D), lambda i: (0, 0)),
        ],
        out_specs=pl.BlockSpec((ROW_BLK, D), lambda i: (i, 0)),
        out_shape=jax.ShapeDtypeStruct((N, D), jnp.float32),
    )

    def run(x, edge_index, W1, b1, W2, b2):
        src = edge_index[0]
        dst = edge_index[1]
        pad = EPAD - E
        srcp = jnp.concatenate([src, jnp.zeros((pad,), jnp.int32)])
        dstp = jnp.concatenate([dst, jnp.full((pad,), N, jnp.int32)])
        src3 = srcp.reshape(NW, NCHUNK, CH)
        dst3 = dstp.reshape(NW, NCHUNK, CH)

        degp = sc_deg(dst3).reshape(NC, NPAD1)
        dinv = lax.rsqrt(degp[0, :N] + degp[1, :N] + 1.0)
        dinv2 = dinv[:, None]

        b1r = b1.reshape(1, D)
        b2r = b2.reshape(1, D)

        y1 = tc_m1(x, W1, dinv2)
        s1 = sc_agg(y1, src3, dst3)
        y2 = tc_m2(s1, y1, dinv2, b1r, W2)
        s2 = sc_agg(y2, src3, dst3)
        return tc_ep(s2, y2, dinv2, b2r)

    return run


@jax.jit
def kernel(x, edge_index, W1, b1, W2, b2):
    return _build()(x, edge_index, W1, b1, W2, b2)


# speedup vs baseline: 1.5151x; 1.5151x over previous
import functools

import jax
import jax.numpy as jnp
from jax import lax
from jax.experimental import pallas as pl
from jax.experimental.pallas import tpu as pltpu
from jax.experimental.pallas import tpu_sc as plsc

N = 10000
D = 128
E = 320000
CH = 128
ROW_BLK = 1000


@functools.lru_cache(maxsize=None)
def _build():
    mesh = plsc.VectorSubcoreMesh(core_axis_name="c", subcore_axis_name="s")
    NC, NS = mesh.num_cores, mesh.num_subcores
    NW = NC * NS
    NCHUNK = -(-E // (NW * CH))
    EPAD = NW * NCHUNK * CH
    ACC_ROWS = ((N + 16 * NS - 1) // (16 * NS)) * (16 * NS)
    RPT = ACC_ROWS // NS
    NFULL = N // RPT
    NREM = N - NFULL * RPT
    NPAD1 = ACC_ROWS
    ZB = NPAD1 // NS

    @functools.partial(
        pl.kernel,
        out_type=jax.ShapeDtypeStruct((NC * NPAD1,), jnp.float32),
        mesh=mesh,
        scratch_types=[
            pltpu.VMEM((NCHUNK, CH), jnp.int32),
            pltpu.VMEM((CH,), jnp.float32),
            pltpu.VMEM((ZB,), jnp.float32),
            pltpu.VMEM_SHARED((NPAD1,), jnp.float32),
        ],
    )
    def sc_deg(dst_hbm, out_hbm, dst_v, ones_v, zb_v, dacc):
        cid = lax.axis_index("c")
        sid = lax.axis_index("s")
        tid = cid * NS + sid
        pltpu.sync_copy(dst_hbm.at[tid], dst_v)

        @pl.loop(0, CH, step=16)
        def _(i):
            ones_v[pl.ds(pl.multiple_of(i, 16), 16)] = jnp.full(
                (16,), 1.0, jnp.float32)

        @pl.loop(0, ZB, step=16)
        def _(i):
            zb_v[pl.ds(pl.multiple_of(i, 16), 16)] = jnp.zeros(
                (16,), jnp.float32)

        pltpu.sync_copy(zb_v, dacc.at[pl.ds(sid * ZB, ZB)])
        plsc.subcore_barrier()

        @pl.loop(0, NCHUNK)
        def _(k):
            pltpu.sync_copy(ones_v, dacc.at[dst_v.at[k]], add=True)

        plsc.subcore_barrier()
        pltpu.sync_copy(dacc.at[pl.ds(sid * ZB, ZB)],
                        out_hbm.at[pl.ds(cid * NPAD1 + sid * ZB, ZB)])

    @functools.partial(
        pl.kernel,
        out_type=jax.ShapeDtypeStruct((NC, ACC_ROWS, D), jnp.float32),
        mesh=mesh,
        scratch_types=[
            pltpu.VMEM((NCHUNK, CH), jnp.int32),
            pltpu.VMEM((NCHUNK, CH), jnp.int32),
            pltpu.VMEM((CH, D), jnp.float32),
            pltpu.VMEM_SHARED((ACC_ROWS, D), jnp.float32),
        ],
    )
    def sc_agg(y_hbm, zeros_hbm, src_hbm, dst_hbm, out_hbm,
               src_v, dst_v, rows_v, acc):
        cid = lax.axis_index("c")
        sid = lax.axis_index("s")
        tid = cid * NS + sid
        pltpu.sync_copy(src_hbm.at[tid], src_v)
        pltpu.sync_copy(dst_hbm.at[tid], dst_v)
        r0 = sid * RPT

        def init_from(src_hbm_ref):
            @pl.when(sid < NFULL)
            def _():
                pltpu.sync_copy(src_hbm_ref.at[pl.ds(r0, RPT)],
                                acc.at[pl.ds(r0, RPT)])

            if NREM:
                @pl.when(sid == NFULL)
                def _():
                    pltpu.sync_copy(src_hbm_ref.at[pl.ds(r0, NREM)],
                                    acc.at[pl.ds(r0, NREM)])

        @pl.when(cid == 0)
        def _():
            init_from(y_hbm)

        @pl.when(cid != 0)
        def _():
            init_from(zeros_hbm)

        plsc.subcore_barrier()

        @pl.loop(0, NCHUNK)
        def _(k):
            pltpu.sync_copy(y_hbm.at[src_v.at[k]], rows_v)
            pltpu.sync_copy(rows_v, acc.at[dst_v.at[k]], add=True)

        plsc.subcore_barrier()
        pltpu.sync_copy(acc.at[pl.ds(r0, RPT)],
                        out_hbm.at[cid, pl.ds(r0, RPT)])

    dot = functools.partial(
        lax.dot_general,
        dimension_numbers=(((1,), (0,)), ((), ())),
        precision=lax.Precision.HIGHEST,
        preferred_element_type=jnp.float32,
    )

    def m1_body(x_ref, w_ref, dinv_ref, o_ref):
        o_ref[...] = dot(x_ref[...], w_ref[...]) * dinv_ref[...]

    tc_m1 = pl.pallas_call(
        m1_body,
        grid=(N // ROW_BLK,),
        in_specs=[
            pl.BlockSpec((ROW_BLK, D), lambda i: (i, 0)),
            pl.BlockSpec((D, D), lambda i: (0, 0)),
            pl.BlockSpec((ROW_BLK, 1), lambda i: (i, 0)),
        ],
        out_specs=pl.BlockSpec((ROW_BLK, D), lambda i: (i, 0)),
        out_shape=jax.ShapeDtypeStruct((N, D), jnp.float32),
    )

    def m2_body(s_ref, dinv_ref, b_ref, w_ref, o_ref):
        h = (s_ref[0] + s_ref[1]) * dinv_ref[...] + b_ref[...]
        h = jnp.maximum(h, 0.0)
        o_ref[...] = dot(h, w_ref[...]) * dinv_ref[...]

    tc_m2 = pl.pallas_call(
        m2_body,
        grid=(N // ROW_BLK,),
        in_specs=[
            pl.BlockSpec((NC, ROW_BLK, D), lambda i: (0, i, 0)),
            pl.BlockSpec((ROW_BLK, 1), lambda i: (i, 0)),
            pl.BlockSpec((1, D), lambda i: (0, 0)),
            pl.BlockSpec((D, D), lambda i: (0, 0)),
        ],
        out_specs=pl.BlockSpec((ROW_BLK, D), lambda i: (i, 0)),
        out_shape=jax.ShapeDtypeStruct((N, D), jnp.float32),
    )

    def ep_body(s_ref, dinv_ref, b_ref, o_ref):
        o_ref[...] = (s_ref[0] + s_ref[1]) * dinv_ref[...] + b_ref[...]

    tc_ep = pl.pallas_call(
        ep_body,
        grid=(N // ROW_BLK,),
        in_specs=[
            pl.BlockSpec((NC, ROW_BLK, D), lambda i: (0, i, 0)),
            pl.BlockSpec((ROW_BLK, 1), lambda i: (i, 0)),
            pl.BlockSpec((1, D), lambda i: (0, 0)),
        ],
        out_specs=pl.BlockSpec((ROW_BLK, D), lambda i: (i, 0)),
        out_shape=jax.ShapeDtypeStruct((N, D), jnp.float32),
    )

    def run(x, edge_index, W1, b1, W2, b2):
        src = edge_index[0]
        dst = edge_index[1]
        pad = EPAD - E
        srcp = jnp.concatenate([src, jnp.zeros((pad,), jnp.int32)])
        dstp = jnp.concatenate([dst, jnp.full((pad,), N, jnp.int32)])
        src3 = srcp.reshape(NW, NCHUNK, CH)
        dst3 = dstp.reshape(NW, NCHUNK, CH)

        degp = sc_deg(dst3).reshape(NC, NPAD1)
        dinv = lax.rsqrt(degp[0, :N] + degp[1, :N] + 1.0)
        dinv2 = dinv[:, None]

        zeros2 = jnp.zeros((N, D), jnp.float32)
        b1r = b1.reshape(1, D)
        b2r = b2.reshape(1, D)

        y1 = tc_m1(x, W1, dinv2)
        s1 = sc_agg(y1, zeros2, src3, dst3)
        y2 = tc_m2(s1, dinv2, b1r, W2)
        s2 = sc_agg(y2, zeros2, src3, dst3)
        return tc_ep(s2, dinv2, b2r)

    return run


@jax.jit
def kernel(x, edge_index, W1, b1, W2, b2):
    return _build()(x, edge_index, W1, b1, W2, b2)


# speedup vs baseline: 1.5251x; 1.0066x over previous
import functools

import jax
import jax.numpy as jnp
from jax import lax
from jax.experimental import pallas as pl
from jax.experimental.pallas import tpu as pltpu
from jax.experimental.pallas import tpu_sc as plsc

N = 10000
D = 128
E = 320000
CH = 128
ROW_BLK = 1000


@functools.lru_cache(maxsize=None)
def _build():
    mesh = plsc.VectorSubcoreMesh(core_axis_name="c", subcore_axis_name="s")
    NC, NS = mesh.num_cores, mesh.num_subcores
    NW = NC * NS
    NCHUNK = -(-E // (NW * CH))
    EPAD = NW * NCHUNK * CH
    ACC_ROWS = ((N + 16 * NS - 1) // (16 * NS)) * (16 * NS)
    RPT = ACC_ROWS // NS
    NFULL = N // RPT
    NREM = N - NFULL * RPT
    NPAD1 = ACC_ROWS
    ZB = NPAD1 // NS

    @functools.partial(
        pl.kernel,
        out_type=jax.ShapeDtypeStruct((NC * NPAD1,), jnp.float32),
        mesh=mesh,
        scratch_types=[
            pltpu.VMEM((NCHUNK, CH), jnp.int32),
            pltpu.VMEM((CH,), jnp.float32),
            pltpu.VMEM((ZB,), jnp.float32),
            pltpu.VMEM_SHARED((NPAD1,), jnp.float32),
        ],
    )
    def sc_deg(dst_hbm, out_hbm, dst_v, ones_v, zb_v, dacc):
        cid = lax.axis_index("c")
        sid = lax.axis_index("s")
        tid = cid * NS + sid
        pltpu.sync_copy(dst_hbm.at[tid], dst_v)

        @pl.loop(0, CH, step=16)
        def _(i):
            ones_v[pl.ds(pl.multiple_of(i, 16), 16)] = jnp.full(
                (16,), 1.0, jnp.float32)

        @pl.loop(0, ZB, step=16)
        def _(i):
            zb_v[pl.ds(pl.multiple_of(i, 16), 16)] = jnp.zeros(
                (16,), jnp.float32)

        pltpu.sync_copy(zb_v, dacc.at[pl.ds(sid * ZB, ZB)])
        plsc.subcore_barrier()

        @pl.loop(0, NCHUNK)
        def _(k):
            pltpu.sync_copy(ones_v, dacc.at[dst_v.at[k]], add=True)

        plsc.subcore_barrier()
        pltpu.sync_copy(dacc.at[pl.ds(sid * ZB, ZB)],
                        out_hbm.at[pl.ds(cid * NPAD1 + sid * ZB, ZB)])

    @functools.partial(
        pl.kernel,
        out_type=jax.ShapeDtypeStruct((NC, ACC_ROWS, D), jnp.float32),
        mesh=mesh,
        scratch_types=[
            pltpu.VMEM((NCHUNK, CH), jnp.int32),
            pltpu.VMEM((NCHUNK, CH), jnp.int32),
            pltpu.VMEM((CH, D), jnp.float32),
            pltpu.VMEM_SHARED((ACC_ROWS, D), jnp.float32),
        ],
    )
    def sc_agg(y_hbm, zeros_hbm, src_hbm, dst_hbm, out_hbm,
               src_v, dst_v, rows_v, acc):
        cid = lax.axis_index("c")
        sid = lax.axis_index("s")
        tid = cid * NS + sid
        pltpu.sync_copy(src_hbm.at[tid], src_v)
        pltpu.sync_copy(dst_hbm.at[tid], dst_v)
        r0 = sid * RPT

        def init_from(src_hbm_ref):
            @pl.when(sid < NFULL)
            def _():
                pltpu.sync_copy(src_hbm_ref.at[pl.ds(r0, RPT)],
                                acc.at[pl.ds(r0, RPT)])

            if NREM:
                @pl.when(sid == NFULL)
                def _():
                    pltpu.sync_copy(src_hbm_ref.at[pl.ds(r0, NREM)],
                                    acc.at[pl.ds(r0, NREM)])

        @pl.when(cid == 0)
        def _():
            init_from(y_hbm)

        @pl.when(cid != 0)
        def _():
            init_from(zeros_hbm)

        plsc.subcore_barrier()

        @pl.loop(0, NCHUNK)
        def _(k):
            pltpu.sync_copy(y_hbm.at[src_v.at[k]], rows_v)
            pltpu.sync_copy(rows_v, acc.at[dst_v.at[k]], add=True)

        plsc.subcore_barrier()
        pltpu.sync_copy(acc.at[pl.ds(r0, RPT)],
                        out_hbm.at[cid, pl.ds(r0, RPT)])

    dot = functools.partial(
        lax.dot_general,
        dimension_numbers=(((1,), (0,)), ((), ())),
        precision=lax.Precision.HIGHEST,
        preferred_element_type=jnp.float32,
    )

    def m1_body(x_ref, w_ref, dinv_ref, o_ref):
        o_ref[...] = dot(x_ref[...], w_ref[...]) * dinv_ref[...]

    tc_m1 = pl.pallas_call(
        m1_body,
        grid=(N // ROW_BLK,),
        in_specs=[
            pl.BlockSpec((ROW_BLK, D), lambda i: (i, 0)),
            pl.BlockSpec((D, D), lambda i: (0, 0)),
            pl.BlockSpec((ROW_BLK, 1), lambda i: (i, 0)),
        ],
        out_specs=pl.BlockSpec((ROW_BLK, D), lambda i: (i, 0)),
        out_shape=jax.ShapeDtypeStruct((N, D), jnp.float32),
    )

    def m2_body(s_ref, dinv_ref, b_ref, w_ref, o_ref):
        h = (s_ref[0] + s_ref[1]) * dinv_ref[...] + b_ref[...]
        h = jnp.maximum(h, 0.0)
        o_ref[...] = dot(h, w_ref[...]) * dinv_ref[...]

    tc_m2 = pl.pallas_call(
        m2_body,
        grid=(N // ROW_BLK,),
        in_specs=[
            pl.BlockSpec((NC, ROW_BLK, D), lambda i: (0, i, 0)),
            pl.BlockSpec((ROW_BLK, 1), lambda i: (i, 0)),
            pl.BlockSpec((1, D), lambda i: (0, 0)),
            pl.BlockSpec((D, D), lambda i: (0, 0)),
        ],
        out_specs=pl.BlockSpec((ROW_BLK, D), lambda i: (i, 0)),
        out_shape=jax.ShapeDtypeStruct((N, D), jnp.float32),
    )

    def ep_body(s_ref, dinv_ref, b_ref, o_ref):
        o_ref[...] = (s_ref[0] + s_ref[1]) * dinv_ref[...] + b_ref[...]

    tc_ep = pl.pallas_call(
        ep_body,
        grid=(N // ROW_BLK,),
        in_specs=[
            pl.BlockSpec((NC, ROW_BLK, D), lambda i: (0, i, 0)),
            pl.BlockSpec((ROW_BLK, 1), lambda i: (i, 0)),
            pl.BlockSpec((1, D), lambda i: (0, 0)),
        ],
        out_specs=pl.BlockSpec((ROW_BLK, D), lambda i: (i, 0)),
        out_shape=jax.ShapeDtypeStruct((N, D), jnp.float32),
    )

    def run(x, edge_index, W1, b1, W2, b2):
        src = edge_index[0]
        dst = edge_index[1]
        pad = EPAD - E
        srcp = jnp.concatenate([src, jnp.zeros((pad,), jnp.int32)])
        spare = ACC_ROWS - N
        pad_dst = N + jnp.arange(pad, dtype=jnp.int32) % spare
        dstp = jnp.concatenate([dst, pad_dst])
        src3 = srcp.reshape(NW, NCHUNK, CH)
        dst3 = dstp.reshape(NW, NCHUNK, CH)

        degp = sc_deg(dst3).reshape(NC, NPAD1)
        dinv = lax.rsqrt(degp[0, :N] + degp[1, :N] + 1.0)
        dinv2 = dinv[:, None]

        zeros2 = jnp.zeros((N, D), jnp.float32)
        b1r = b1.reshape(1, D)
        b2r = b2.reshape(1, D)

        y1 = tc_m1(x, W1, dinv2)
        s1 = sc_agg(y1, zeros2, src3, dst3)
        y2 = tc_m2(s1, dinv2, b1r, W2)
        s2 = sc_agg(y2, zeros2, src3, dst3)
        return tc_ep(s2, dinv2, b2r)

    return run


@jax.jit
def kernel(x, edge_index, W1, b1, W2, b2):
    return _build()(x, edge_index, W1, b1, W2, b2)
